# baseline (speedup 1.0000x reference)
import jax
import jax.numpy as jnp
from jax import lax
from jax.experimental import pallas as pl
from jax.experimental.pallas import tpu as pltpu
from jax.experimental.pallas import tpu_sc as plsc

N_NODES = 10000
N_PAD = 10112
D_CLS = 64
NC, NS = 2, 16
NW = NC * NS
CH = 200
KJ = 50
E_W = CH * KJ
E_PAD = NW * E_W
RPT = N_PAD // NS
ZB = RPT // 4
DUMMY = N_NODES

_sc_mesh = plsc.VectorSubcoreMesh(core_axis_name="c", subcore_axis_name="s")
_sc_params = pltpu.CompilerParams(use_tc_tiling_on_sc=False)


def _sc_degree_body(col_hbm, ones_hbm, zeros_hbm, degp_hbm,
                    colv, ones_v, bounce, deg_sh, ssem):
    c = lax.axis_index("c")
    s = lax.axis_index("s")
    wid = c * NS + s
    pltpu.sync_copy(col_hbm.at[wid], colv)
    pltpu.sync_copy(ones_hbm, ones_v)
    pltpu.sync_copy(zeros_hbm.at[pl.ds(s * RPT, RPT)], bounce)
    pltpu.sync_copy(bounce, deg_sh.at[pl.ds(s * RPT, RPT)])
    plsc.subcore_barrier()

    def fire(j, carry):
        pltpu.make_async_copy(
            ones_v, deg_sh.at[colv.at[j]], ssem).start(add=True)
        return carry

    def drain(j, carry):
        pltpu.make_async_copy(
            ones_v, deg_sh.at[colv.at[j]], ssem).wait()
        return carry

    lax.fori_loop(0, KJ, fire, 0)
    lax.fori_loop(0, KJ, drain, 0)
    plsc.subcore_barrier()
    pltpu.sync_copy(deg_sh.at[pl.ds(s * RPT, RPT)], bounce)
    pltpu.sync_copy(bounce, degp_hbm.at[c].at[pl.ds(s * RPT, RPT)])


_deg_call = pl.kernel(
    _sc_degree_body,
    out_type=jax.ShapeDtypeStruct((NC, N_PAD, 8), jnp.float32),
    mesh=_sc_mesh,
    scratch_types=[
        pltpu.VMEM((KJ, CH), jnp.int32),
        pltpu.VMEM((CH, 8), jnp.float32),
        pltpu.VMEM((RPT, 8), jnp.float32),
        pltpu.VMEM_SHARED((N_PAD, 8), jnp.float32),
        pltpu.SemaphoreType.DMA,
    ],
    compiler_params=_sc_params,
)


def _sc_hop_body(row_hbm, col_hbm, t_hbm, zeros_hbm, out_hbm,
                 rowv, colv, buf, bounce, u_sh, gsem, ssem):
    c = lax.axis_index("c")
    s = lax.axis_index("s")
    wid = c * NS + s
    pltpu.sync_copy(row_hbm.at[wid], rowv)
    pltpu.make_async_copy(
        t_hbm.at[rowv.at[0]], buf.at[0], gsem).start()
    pltpu.sync_copy(col_hbm.at[wid], colv)
    for z in range(4):
        pltpu.sync_copy(zeros_hbm.at[pl.ds(s * RPT + z * ZB, ZB)], bounce)
        pltpu.sync_copy(bounce, u_sh.at[pl.ds(s * RPT + z * ZB, ZB)])
    plsc.subcore_barrier()

    def step(j, carry):
        p = lax.rem(j, 3)
        pltpu.make_async_copy(
            t_hbm.at[rowv.at[j]], buf.at[p], gsem).wait()

        @pl.when(j >= 2)
        def _():
            po = lax.rem(j - 2, 3)
            pltpu.make_async_copy(
                buf.at[po], u_sh.at[colv.at[j - 2]], ssem).wait()

        @pl.when(j + 1 < KJ)
        def _():
            pn = lax.rem(j + 1, 3)
            pltpu.make_async_copy(
                t_hbm.at[rowv.at[j + 1]], buf.at[pn], gsem).start()

        pltpu.make_async_copy(
            buf.at[p], u_sh.at[colv.at[j]], ssem).start(add=True)
        return carry

    lax.fori_loop(0, KJ, step, 0)
    for j in (KJ - 2, KJ - 1):
        pltpu.make_async_copy(
            buf.at[j % 3], u_sh.at[colv.at[j]], ssem).wait()
    plsc.subcore_barrier()
    for z in range(4):
        pltpu.sync_copy(u_sh.at[pl.ds(s * RPT + z * ZB, ZB)], bounce)
        pltpu.sync_copy(bounce,
                        out_hbm.at[c].at[pl.ds(s * RPT + z * ZB, ZB)])


_hop_call = pl.kernel(
    _sc_hop_body,
    out_type=jax.ShapeDtypeStruct((NC, N_PAD, D_CLS), jnp.float32),
    mesh=_sc_mesh,
    scratch_types=[
        pltpu.VMEM((KJ, CH), jnp.int32),
        pltpu.VMEM((KJ, CH), jnp.int32),
        pltpu.VMEM((3, CH, D_CLS), jnp.float32),
        pltpu.VMEM((ZB, D_CLS), jnp.float32),
        pltpu.VMEM_SHARED((N_PAD, D_CLS), jnp.float32),
        pltpu.SemaphoreType.DMA,
        pltpu.SemaphoreType.DMA,
    ],
    compiler_params=_sc_params,
)


def _tc_pre_body(x_ref, wt_ref, degp_ref, t1_ref, dis_ref):
    z = jnp.dot(x_ref[...], wt_ref[...], preferred_element_type=jnp.float32)
    deg = degp_ref[0, :, 0:1] + degp_ref[1, :, 0:1] + 1.0
    dis = lax.rsqrt(deg)
    dis_ref[...] = dis
    t1_ref[...] = z * dis


def _tc_pre(x_pad, wt, degp):
    return pl.pallas_call(
        _tc_pre_body,
        out_shape=(
            jax.ShapeDtypeStruct((N_PAD, D_CLS), jnp.float32),
            jax.ShapeDtypeStruct((N_PAD, 1), jnp.float32),
        ),
    )(x_pad, wt, degp)


def _tc_mid_body(up_ref, t_ref, dis_ref, out_ref):
    dis = dis_ref[...]
    u = up_ref[0] + up_ref[1] + t_ref[...]
    out_ref[...] = u * (dis * dis)


def _tc_mid(up, t, dis):
    return pl.pallas_call(
        _tc_mid_body,
        out_shape=jax.ShapeDtypeStruct((N_PAD, D_CLS), jnp.float32),
    )(up, t, dis)


def _tc_final_body(up_ref, t_ref, dis_ref, b_ref, out_ref):
    h = (up_ref[0] + up_ref[1] + t_ref[...]) * dis_ref[...] + b_ref[...]
    m = jnp.max(h, axis=1, keepdims=True)
    lse = jnp.log(jnp.sum(jnp.exp(h - m), axis=1, keepdims=True)) + m
    out_ref[...] = h - lse


def _tc_final(up, t, dis, b2):
    return pl.pallas_call(
        _tc_final_body,
        out_shape=jax.ShapeDtypeStruct((N_PAD, D_CLS), jnp.float32),
    )(up, t, dis, b2)


def kernel(x, edge_index, W, b):
    row2 = edge_index[0].astype(jnp.int32).reshape(NW, KJ, CH)
    col2 = edge_index[1].astype(jnp.int32).reshape(NW, KJ, CH)
    x_pad = jnp.pad(x, ((0, N_PAD - x.shape[0]), (0, 0)))
    wt = W.T.astype(jnp.float32)
    zeros16 = jnp.zeros((N_PAD, 8), jnp.float32)
    zeros64 = jnp.zeros((N_PAD, D_CLS), jnp.float32)
    ones16 = jnp.ones((CH, 8), jnp.float32)

    degp = _deg_call(col2, ones16, zeros16)
    t1, dis = _tc_pre(x_pad, wt, degp)
    u1 = _hop_call(row2, col2, t1, zeros64)
    t2 = _tc_mid(u1, t1, dis)
    u2 = _hop_call(row2, col2, t2, zeros64)
    out = _tc_final(u2, t2, dis, b.reshape(1, D_CLS))
    return out[:N_NODES]


# speedup vs baseline: 1.0919x; 1.0919x over previous
import jax
import jax.numpy as jnp
from jax import lax
from jax.experimental import pallas as pl
from jax.experimental.pallas import tpu as pltpu
from jax.experimental.pallas import tpu_sc as plsc

N_NODES = 10000
N_PAD = 10112
D_CLS = 64
NC, NS = 2, 16
NW = NC * NS
CH = 400
KJ = 25
E_W = CH * KJ
E_PAD = NW * E_W
RPT = N_PAD // NS
ZB = RPT // 4
DUMMY = N_NODES

_sc_mesh = plsc.VectorSubcoreMesh(core_axis_name="c", subcore_axis_name="s")
_sc_params = pltpu.CompilerParams(use_tc_tiling_on_sc=False)


def _sc_degree_body(col_hbm, ones_hbm, zeros_hbm, degp_hbm,
                    colv, ones_v, bounce, deg_sh, ssem):
    c = lax.axis_index("c")
    s = lax.axis_index("s")
    wid = c * NS + s
    pltpu.sync_copy(col_hbm.at[wid], colv)
    pltpu.sync_copy(ones_hbm, ones_v)
    pltpu.sync_copy(zeros_hbm.at[pl.ds(s * RPT, RPT)], bounce)
    pltpu.sync_copy(bounce, deg_sh.at[pl.ds(s * RPT, RPT)])
    plsc.subcore_barrier()

    def fire(j, carry):
        pltpu.make_async_copy(
            ones_v, deg_sh.at[colv.at[j]], ssem).start(add=True)
        return carry

    def drain(j, carry):
        pltpu.make_async_copy(
            ones_v, deg_sh.at[colv.at[j]], ssem).wait()
        return carry

    lax.fori_loop(0, KJ, fire, 0)
    lax.fori_loop(0, KJ, drain, 0)
    plsc.subcore_barrier()
    pltpu.sync_copy(deg_sh.at[pl.ds(s * RPT, RPT)], bounce)
    pltpu.sync_copy(bounce, degp_hbm.at[c].at[pl.ds(s * RPT, RPT)])


_deg_call = pl.kernel(
    _sc_degree_body,
    out_type=jax.ShapeDtypeStruct((NC, N_PAD, 8), jnp.float32),
    mesh=_sc_mesh,
    scratch_types=[
        pltpu.VMEM((KJ, CH), jnp.int32),
        pltpu.VMEM((CH, 8), jnp.float32),
        pltpu.VMEM((RPT, 8), jnp.float32),
        pltpu.VMEM_SHARED((N_PAD, 8), jnp.float32),
        pltpu.SemaphoreType.DMA,
    ],
    compiler_params=_sc_params,
)


def _sc_hop_body(row_hbm, col_hbm, t_hbm, zeros_hbm, out_hbm,
                 rowv, colv, buf, bounce, u_sh, gsem):
    c = lax.axis_index("c")
    s = lax.axis_index("s")
    wid = c * NS + s
    pltpu.sync_copy(row_hbm.at[wid], rowv)
    pltpu.make_async_copy(
        t_hbm.at[rowv.at[0]], buf.at[0], gsem).start()
    pltpu.sync_copy(col_hbm.at[wid], colv)
    for z in range(4):
        pltpu.sync_copy(zeros_hbm.at[pl.ds(s * RPT + z * ZB, ZB)], bounce)
        pltpu.sync_copy(bounce, u_sh.at[pl.ds(s * RPT + z * ZB, ZB)])
    plsc.subcore_barrier()

    def step(j, carry):
        p = lax.rem(j, 2)
        pltpu.make_async_copy(
            t_hbm.at[rowv.at[j]], buf.at[p], gsem).wait()

        @pl.when(j + 1 < KJ)
        def _():
            pn = lax.rem(j + 1, 2)
            pltpu.make_async_copy(
                t_hbm.at[rowv.at[j + 1]], buf.at[pn], gsem).start()

        pltpu.sync_copy(buf.at[p], u_sh.at[colv.at[j]], add=True)
        return carry

    lax.fori_loop(0, KJ, step, 0)
    plsc.subcore_barrier()
    for z in range(4):
        pltpu.sync_copy(u_sh.at[pl.ds(s * RPT + z * ZB, ZB)], bounce)
        pltpu.sync_copy(bounce,
                        out_hbm.at[c].at[pl.ds(s * RPT + z * ZB, ZB)])


_hop_call = pl.kernel(
    _sc_hop_body,
    out_type=jax.ShapeDtypeStruct((NC, N_PAD, D_CLS), jnp.float32),
    mesh=_sc_mesh,
    scratch_types=[
        pltpu.VMEM((KJ, CH), jnp.int32),
        pltpu.VMEM((KJ, CH), jnp.int32),
        pltpu.VMEM((2, CH, D_CLS), jnp.float32),
        pltpu.VMEM((ZB, D_CLS), jnp.float32),
        pltpu.VMEM_SHARED((N_PAD, D_CLS), jnp.float32),
        pltpu.SemaphoreType.DMA,
    ],
    compiler_params=_sc_params,
)


def _tc_pre_body(x_ref, wt_ref, degp_ref, t1_ref, dis_ref):
    z = jnp.dot(x_ref[...], wt_ref[...], preferred_element_type=jnp.float32)
    deg = degp_ref[0, :, 0:1] + degp_ref[1, :, 0:1] + 1.0
    dis = lax.rsqrt(deg)
    dis_ref[...] = dis
    t1_ref[...] = z * dis


def _tc_pre(x_pad, wt, degp):
    return pl.pallas_call(
        _tc_pre_body,
        out_shape=(
            jax.ShapeDtypeStruct((N_PAD, D_CLS), jnp.float32),
            jax.ShapeDtypeStruct((N_PAD, 1), jnp.float32),
        ),
    )(x_pad, wt, degp)


def _tc_mid_body(up_ref, t_ref, dis_ref, out_ref):
    dis = dis_ref[...]
    u = up_ref[0] + up_ref[1] + t_ref[...]
    out_ref[...] = u * (dis * dis)


def _tc_mid(up, t, dis):
    return pl.pallas_call(
        _tc_mid_body,
        out_shape=jax.ShapeDtypeStruct((N_PAD, D_CLS), jnp.float32),
    )(up, t, dis)


def _tc_final_body(up_ref, t_ref, dis_ref, b_ref, out_ref):
    h = (up_ref[0] + up_ref[1] + t_ref[...]) * dis_ref[...] + b_ref[...]
    m = jnp.max(h, axis=1, keepdims=True)
    lse = jnp.log(jnp.sum(jnp.exp(h - m), axis=1, keepdims=True)) + m
    out_ref[...] = h - lse


def _tc_final(up, t, dis, b2):
    return pl.pallas_call(
        _tc_final_body,
        out_shape=jax.ShapeDtypeStruct((N_PAD, D_CLS), jnp.float32),
    )(up, t, dis, b2)


def kernel(x, edge_index, W, b):
    row2 = edge_index[0].astype(jnp.int32).reshape(NW, KJ, CH)
    col2 = edge_index[1].astype(jnp.int32).reshape(NW, KJ, CH)
    x_pad = jnp.pad(x, ((0, N_PAD - x.shape[0]), (0, 0)))
    wt = W.T.astype(jnp.float32)
    zeros16 = jnp.zeros((N_PAD, 8), jnp.float32)
    zeros64 = jnp.zeros((N_PAD, D_CLS), jnp.float32)
    ones16 = jnp.ones((CH, 8), jnp.float32)

    degp = _deg_call(col2, ones16, zeros16)
    t1, dis = _tc_pre(x_pad, wt, degp)
    u1 = _hop_call(row2, col2, t1, zeros64)
    t2 = _tc_mid(u1, t1, dis)
    u2 = _hop_call(row2, col2, t2, zeros64)
    out = _tc_final(u2, t2, dis, b.reshape(1, D_CLS))
    return out[:N_NODES]
